# baseline (speedup 1.0000x reference)
import functools

import jax
import jax.numpy as jnp
from jax import lax
from jax.experimental import pallas as pl
from jax.experimental.pallas import tpu as pltpu
from jax.experimental.pallas import tpu_sc as plsc

NUM_CLASSES = 1000000
D = 64
B = 16384

_info = plsc.get_sparse_core_info()
NC, NS, L = _info.num_cores, _info.num_subcores, _info.num_lanes
NW = NC * NS
B_PER_W = B // NW

_mesh = plsc.VectorSubcoreMesh(core_axis_name="c", subcore_axis_name="s")


@functools.partial(
    pl.kernel,
    mesh=_mesh,
    out_type=jax.ShapeDtypeStruct((B * D,), jnp.float32),
    scratch_types=[
        pltpu.VMEM((B_PER_W,), jnp.int32),
        pltpu.VMEM((B_PER_W * D,), jnp.float32),
        pltpu.SemaphoreType.DMA,
    ],
)
def _gather_kernel(idx_hbm, table_hbm, out_hbm, idx_v, rows_v, sem):
    wid = lax.axis_index("s") * NC + lax.axis_index("c")
    base = wid * B_PER_W
    pltpu.sync_copy(idx_hbm.at[pl.ds(base, B_PER_W)], idx_v)

    n_groups = B_PER_W // L
    pending = []
    for g in range(n_groups):
        v = idx_v[pl.ds(g * L, L)]
        offv = lax.mul(v, jnp.int32(D))
        fired = [
            pltpu.async_copy(
                table_hbm.at[pl.ds(pl.multiple_of(offv[j], D), D)],
                rows_v.at[pl.ds((g * L + j) * D, D)], sem)
            for j in range(L)
        ]
        for c in pending:
            c.wait()
        pending = fired
    for c in pending:
        c.wait()
    pltpu.sync_copy(rows_v, out_hbm.at[pl.ds(base * D, B_PER_W * D)])


def kernel(input, embedding_table):
    flat = _gather_kernel(input.astype(jnp.int32),
                          embedding_table.reshape(NUM_CLASSES * D))
    return flat.reshape(B, D)


# speedup vs baseline: 1.6995x; 1.6995x over previous
import functools

import jax
import jax.numpy as jnp
from jax import lax
from jax.experimental import pallas as pl
from jax.experimental.pallas import tpu as pltpu
from jax.experimental.pallas import tpu_sc as plsc

NUM_CLASSES = 1000000
D = 64
B = 16384

_info = plsc.get_sparse_core_info()
NC, NS, L = _info.num_cores, _info.num_subcores, _info.num_lanes
NW = NC * NS
B_PER_W = B // NW

_mesh = plsc.VectorSubcoreMesh(core_axis_name="c", subcore_axis_name="s")


@functools.partial(
    pl.kernel,
    mesh=_mesh,
    out_type=jax.ShapeDtypeStruct((B, D), jnp.float32),
    compiler_params=pltpu.CompilerParams(needs_layout_passes=False),
    scratch_types=[
        pltpu.VMEM((B_PER_W,), jnp.int32),
        pltpu.VMEM((B_PER_W, D), jnp.float32),
        pltpu.SemaphoreType.DMA,
    ],
)
def _gather_kernel(idx_hbm, table_hbm, out_hbm, idx_v, rows_v, sem):
    wid = lax.axis_index("s") * NC + lax.axis_index("c")
    base = wid * B_PER_W
    pltpu.sync_copy(idx_hbm.at[pl.ds(base, B_PER_W)], idx_v)

    n_groups = B_PER_W // L
    pending = []
    for g in range(n_groups):
        v = idx_v[pl.ds(g * L, L)]
        fired = [
            pltpu.async_copy(table_hbm.at[v[j]],
                             rows_v.at[g * L + j], sem)
            for j in range(L)
        ]
        for c in pending:
            c.wait()
        pending = fired
    for c in pending:
        c.wait()
    pltpu.sync_copy(rows_v, out_hbm.at[pl.ds(base, B_PER_W)])


def kernel(input, embedding_table):
    return _gather_kernel(input.astype(jnp.int32), embedding_table)


# speedup vs baseline: 2.4967x; 1.4691x over previous
import functools

import jax
import jax.numpy as jnp
from jax import lax
from jax.experimental import pallas as pl
from jax.experimental.pallas import tpu as pltpu
from jax.experimental.pallas import tpu_sc as plsc

NUM_CLASSES = 1000000
D = 64
B = 16384
TROWS = 8

_info = plsc.get_sparse_core_info()
NC, NS, L = _info.num_cores, _info.num_subcores, _info.num_lanes
NW = NC * NS
B_PER_W = B // NW

_mesh = plsc.VectorSubcoreMesh(core_axis_name="c", subcore_axis_name="s")


@functools.partial(
    pl.kernel,
    mesh=_mesh,
    out_type=jax.ShapeDtypeStruct((B, D), jnp.float32),
    scratch_types=[
        pltpu.VMEM((B_PER_W,), jnp.int32),
        pltpu.VMEM((B_PER_W, D), jnp.float32),
        pltpu.SemaphoreType.DMA,
    ],
)
def _gather_kernel(idx_hbm, table_hbm, out_hbm, idx_v, rows_v, sem):
    wid = lax.axis_index("s") * NC + lax.axis_index("c")
    base = wid * B_PER_W
    pltpu.sync_copy(idx_hbm.at[pl.ds(base, B_PER_W)], idx_v)

    copies = []
    for g in range(B_PER_W // L):
        v = idx_v[pl.ds(g * L, L)]
        tidv = lax.shift_right_logical(v, 3)
        rv = lax.bitwise_and(v, jnp.int32(TROWS - 1))
        copies.extend(
            pltpu.async_copy(table_hbm.at[tidv[j], rv[j]],
                             rows_v.at[g * L + j], sem)
            for j in range(L)
        )
    for c in copies:
        c.wait()
    pltpu.sync_copy(rows_v, out_hbm.at[pl.ds(base, B_PER_W)])


def kernel(input, embedding_table):
    table3d = embedding_table.reshape(NUM_CLASSES // TROWS, TROWS, D)
    return _gather_kernel(input.astype(jnp.int32), table3d)


# speedup vs baseline: 2.5024x; 1.0023x over previous
import functools

import jax
import jax.numpy as jnp
from jax import lax
from jax.experimental import pallas as pl
from jax.experimental.pallas import tpu as pltpu
from jax.experimental.pallas import tpu_sc as plsc

NUM_CLASSES = 1000000
D = 64
B = 16384
TROWS = 8

_info = plsc.get_sparse_core_info()
NC, NS, L = _info.num_cores, _info.num_subcores, _info.num_lanes
NW = NC * NS
B_PER_W = B // NW

_mesh = plsc.VectorSubcoreMesh(core_axis_name="c", subcore_axis_name="s")


@functools.partial(
    pl.kernel,
    mesh=_mesh,
    out_type=jax.ShapeDtypeStruct((B, D), jnp.float32),
    scratch_types=[
        pltpu.VMEM((B_PER_W,), jnp.int32),
        pltpu.VMEM((B_PER_W, D), jnp.float32),
        pltpu.SemaphoreType.DMA,
        pltpu.SemaphoreType.DMA,
    ],
)
def _gather_kernel(idx_hbm, table_hbm, out_hbm, idx_v, rows_v, sem, osem):
    wid = lax.axis_index("s") * NC + lax.axis_index("c")
    base = wid * B_PER_W
    pltpu.sync_copy(idx_hbm.at[pl.ds(base, B_PER_W)], idx_v)

    copies = []
    for g in range(B_PER_W // L):
        v = idx_v[pl.ds(g * L, L)]
        tidv = lax.shift_right_logical(v, 3)
        rv = lax.bitwise_and(v, jnp.int32(TROWS - 1))
        copies.extend(
            pltpu.async_copy(table_hbm.at[tidv[j], rv[j]],
                             rows_v.at[g * L + j], sem)
            for j in range(L)
        )
    BLK = B_PER_W // 4
    out_copies = []
    for blk in range(4):
        for c in copies[blk * BLK:(blk + 1) * BLK]:
            c.wait()
        out_copies.append(pltpu.async_copy(
            rows_v.at[pl.ds(blk * BLK, BLK)],
            out_hbm.at[pl.ds(base + blk * BLK, BLK)], osem))
    for c in out_copies:
        c.wait()


def kernel(input, embedding_table):
    table3d = embedding_table.reshape(NUM_CLASSES // TROWS, TROWS, D)
    return _gather_kernel(input.astype(jnp.int32), table3d)
